# baseline (speedup 1.0000x reference)
import functools

import jax
import jax.numpy as jnp
from jax import lax
from jax.experimental import pallas as pl
from jax.experimental.pallas import tpu as pltpu
from jax.experimental.pallas import tpu_sc as plsc

N = 4096
E = 16384
G = 128
FP = 560

_NC = 2
_NS = 16
_NW = _NC * _NS
_ROWS_W = N // _NW
_CH = 16

_mesh = functools.partial(
    plsc.VectorSubcoreMesh, core_axis_name="c", subcore_axis_name="s")


def _build_adj_body(edge_hbm, zeros_hbm, a_hbm, src_v, dst_v, chunk_v):
    wid = lax.axis_index("s") * _NC + lax.axis_index("c")
    row0 = wid * _ROWS_W
    pltpu.sync_copy(edge_hbm.at[0], src_v)
    pltpu.sync_copy(edge_hbm.at[1], dst_v)
    ones = jnp.full((16,), 1.0, jnp.float32)
    for c in range(_ROWS_W // _CH):
        r0 = row0 + c * _CH
        pltpu.sync_copy(zeros_hbm, chunk_v)

        def body(j, carry):
            d = dst_v[pl.ds(j * 16, 16)]
            s = src_v[pl.ds(j * 16, 16)]
            rel = d - r0
            m = (rel >= 0) & (rel < _CH)
            idx = jnp.where(m, rel * N + s, 0)
            plsc.store_scatter(chunk_v, [idx], ones, mask=m)
            return carry

        lax.fori_loop(0, E // 16, body, 0)
        pltpu.sync_copy(chunk_v, a_hbm.at[pl.ds(r0 * N, _CH * N)])


def _build_adj(edge_index, zeros_chunk):
    return pl.kernel(
        _build_adj_body,
        out_type=jax.ShapeDtypeStruct((N * N,), jnp.float32),
        mesh=_mesh(),
        compiler_params=pltpu.CompilerParams(needs_layout_passes=False),
        scratch_types=[
            pltpu.VMEM((E,), jnp.int32),
            pltpu.VMEM((E,), jnp.int32),
            pltpu.VMEM((_CH * N,), jnp.float32),
        ],
    )(edge_index, zeros_chunk)


def _segmax_body(concat_hbm, batch_hbm, neginf_hbm, part_hbm,
                 pooled_v, slab_v, batch_v):
    wid = lax.axis_index("s") * _NC + lax.axis_index("c")
    n0 = wid * _ROWS_W
    pltpu.sync_copy(neginf_hbm, pooled_v)
    pltpu.sync_copy(batch_hbm.at[pl.ds(n0, _ROWS_W)],
                    batch_v.at[pl.ds(0, _ROWS_W)])
    for half in range(2):
        pltpu.sync_copy(concat_hbm.at[pl.ds(n0 + half * 64, 64)], slab_v)

        def body(n, carry):
            g = batch_v[pl.ds(half * 64 + n, 16)][0]
            for f in range(FP // 16):
                sl = slab_v[n, pl.ds(f * 16, 16)]
                pv = pooled_v[g, pl.ds(f * 16, 16)]
                pooled_v[g, pl.ds(f * 16, 16)] = jnp.maximum(pv, sl)
            return carry

        lax.fori_loop(0, 64, body, 0)
    pltpu.sync_copy(pooled_v, part_hbm.at[wid])


def _segmax(concat, batch, neginf):
    return pl.kernel(
        _segmax_body,
        out_type=jax.ShapeDtypeStruct((_NW, G, FP), jnp.float32),
        mesh=_mesh(),
        scratch_types=[
            pltpu.VMEM((G, FP), jnp.float32),
            pltpu.VMEM((64, FP), jnp.float32),
            pltpu.VMEM((_ROWS_W + 16,), jnp.int32),
        ],
    )(concat, batch, neginf)


_BM = 256


def _prep_body(a_ref, abf_ref, dinv_ref, omd_ref):
    i = pl.program_id(0)
    a = a_ref[...]
    rowsum = jnp.sum(a, axis=1, keepdims=True)
    r = lax.broadcasted_iota(jnp.int32, a.shape, 0) + i * _BM
    c = lax.broadcasted_iota(jnp.int32, a.shape, 1)
    diag = jnp.sum(jnp.where(r == c, a, 0.0), axis=1, keepdims=True)
    deg = rowsum + 1.0 - diag
    dinv_ref[...] = lax.rsqrt(deg)
    omd_ref[...] = 1.0 - diag
    abf_ref[...] = a.astype(jnp.bfloat16)


def _prep(a):
    return pl.pallas_call(
        _prep_body,
        grid=(N // _BM,),
        in_specs=[pl.BlockSpec((_BM, N), lambda i: (i, 0))],
        out_specs=[
            pl.BlockSpec((_BM, N), lambda i: (i, 0)),
            pl.BlockSpec((_BM, 1), lambda i: (i, 0)),
            pl.BlockSpec((_BM, 1), lambda i: (i, 0)),
        ],
        out_shape=[
            jax.ShapeDtypeStruct((N, N), jnp.bfloat16),
            jax.ShapeDtypeStruct((N, 1), jnp.float32),
            jax.ShapeDtypeStruct((N, 1), jnp.float32),
        ],
    )(a)


def _stats_body(a_ref, dinv_ref, omd_ref):
    i = pl.program_id(0)
    a = a_ref[...].astype(jnp.float32)
    rowsum = jnp.sum(a, axis=1, keepdims=True)
    r = lax.broadcasted_iota(jnp.int32, a.shape, 0) + i * _BM
    c = lax.broadcasted_iota(jnp.int32, a.shape, 1)
    diag = jnp.sum(jnp.where(r == c, a, 0.0), axis=1, keepdims=True)
    deg = rowsum + 1.0 - diag
    dinv_ref[...] = lax.rsqrt(deg)
    omd_ref[...] = 1.0 - diag


def _stats(a_bf):
    return pl.pallas_call(
        _stats_body,
        grid=(N // _BM,),
        in_specs=[pl.BlockSpec((_BM, N), lambda i: (i, 0))],
        out_specs=[
            pl.BlockSpec((_BM, 1), lambda i: (i, 0)),
            pl.BlockSpec((_BM, 1), lambda i: (i, 0)),
        ],
        out_shape=[
            jax.ShapeDtypeStruct((N, 1), jnp.float32),
            jax.ShapeDtypeStruct((N, 1), jnp.float32),
        ],
    )(a_bf)


_MM_BM, _MM_BN, _MM_BK = 1024, 1024, 512


def _binmm_body(x_ref, y_ref, o_ref, acc_ref):
    k = pl.program_id(2)
    p = jnp.dot(x_ref[...], y_ref[...], preferred_element_type=jnp.float32)

    @pl.when(k == 0)
    def _():
        acc_ref[...] = p

    @pl.when(k > 0)
    def _():
        acc_ref[...] += p

    @pl.when(k == pl.num_programs(2) - 1)
    def _():
        o_ref[...] = (acc_ref[...] > 0.0).astype(jnp.bfloat16)


def _binmm(x_bf, y_bf):
    return pl.pallas_call(
        _binmm_body,
        grid=(N // _MM_BM, N // _MM_BN, N // _MM_BK),
        in_specs=[
            pl.BlockSpec((_MM_BM, _MM_BK), lambda i, j, k: (i, k)),
            pl.BlockSpec((_MM_BK, _MM_BN), lambda i, j, k: (k, j)),
        ],
        out_specs=pl.BlockSpec((_MM_BM, _MM_BN), lambda i, j, k: (i, j)),
        out_shape=jax.ShapeDtypeStruct((N, N), jnp.bfloat16),
        scratch_shapes=[pltpu.VMEM((_MM_BM, _MM_BN), jnp.float32)],
    )(x_bf, y_bf)


def _linear_body(h_ref, w_ref, dinv_ref, o_ref):
    o_ref[...] = (
        jnp.dot(h_ref[...], w_ref[...], preferred_element_type=jnp.float32)
        * dinv_ref[...])


def _linear(h, w, dinv):
    fin, fout = w.shape
    return pl.pallas_call(
        _linear_body,
        grid=(N // _BM,),
        in_specs=[
            pl.BlockSpec((_BM, fin), lambda i: (i, 0)),
            pl.BlockSpec((fin, fout), lambda i: (0, 0)),
            pl.BlockSpec((_BM, 1), lambda i: (i, 0)),
        ],
        out_specs=pl.BlockSpec((_BM, fout), lambda i: (i, 0)),
        out_shape=jax.ShapeDtypeStruct((N, fout), jnp.float32),
    )(h, w, dinv)


def _prop_body(a_ref, zs_ref, dinv_ref, omd_ref, b_ref, o_ref):
    i = pl.program_id(0)
    a = a_ref[...].astype(jnp.float32)
    r = lax.broadcasted_iota(jnp.int32, a.shape, 0) + i * _BM
    c = lax.broadcasted_iota(jnp.int32, a.shape, 1)
    a = a + jnp.where(r == c, omd_ref[...], 0.0)
    acc = jnp.dot(a, zs_ref[...], preferred_element_type=jnp.float32)
    o_ref[...] = jnp.maximum(dinv_ref[...] * acc + b_ref[...], 0.0)


def _prop(a_bf, zs, dinv, omd, b):
    f = zs.shape[1]
    return pl.pallas_call(
        _prop_body,
        grid=(N // _BM,),
        in_specs=[
            pl.BlockSpec((_BM, N), lambda i: (i, 0)),
            pl.BlockSpec((N, f), lambda i: (0, 0)),
            pl.BlockSpec((_BM, 1), lambda i: (i, 0)),
            pl.BlockSpec((_BM, 1), lambda i: (i, 0)),
            pl.BlockSpec((1, f), lambda i: (0, 0)),
        ],
        out_specs=pl.BlockSpec((_BM, f), lambda i: (i, 0)),
        out_shape=jax.ShapeDtypeStruct((N, f), jnp.float32),
    )(a_bf, zs, dinv, omd, b)


def _head_body(p_ref, w1_ref, b1_ref, w2_ref, b2_ref, o_ref, acc_ref):
    p = pl.program_id(0)
    blk = p_ref[0]

    @pl.when(p == 0)
    def _():
        acc_ref[...] = blk

    @pl.when(p > 0)
    def _():
        acc_ref[...] = jnp.maximum(acc_ref[...], blk)

    @pl.when(p == pl.num_programs(0) - 1)
    def _():
        t = jnp.dot(acc_ref[...], w1_ref[...],
                    preferred_element_type=jnp.float32) + b1_ref[...]
        t = jnp.maximum(t, 0.0)
        o_ref[...] = jnp.dot(t, w2_ref[...],
                             preferred_element_type=jnp.float32) + b2_ref[...]


def _head(partials, w1p, b1, w2, b2):
    return pl.pallas_call(
        _head_body,
        grid=(_NW,),
        in_specs=[
            pl.BlockSpec((1, G, FP), lambda p: (p, 0, 0)),
            pl.BlockSpec((FP, 512), lambda p: (0, 0)),
            pl.BlockSpec((1, 512), lambda p: (0, 0)),
            pl.BlockSpec((512, G), lambda p: (0, 0)),
            pl.BlockSpec((1, G), lambda p: (0, 0)),
        ],
        out_specs=pl.BlockSpec((G, G), lambda p: (0, 0)),
        out_shape=jax.ShapeDtypeStruct((G, G), jnp.float32),
        scratch_shapes=[pltpu.VMEM((G, FP), jnp.float32)],
    )(partials, w1p, b1, w2, b2)


def kernel(x, edge_index, batch, W1_1, b1_1, W1_2, b1_2, W1_3, b1_3,
           W2_1, b2_1, W2_2, b2_2, W3_1, b3_1, Wg1, bg1, Wg2, bg2):
    zeros_chunk = jnp.zeros((_CH * N,), jnp.float32)
    neginf = jnp.full((G, FP), -jnp.inf, jnp.float32)

    A = _build_adj(edge_index, zeros_chunk).reshape(N, N)
    A_bf, dinv1, omd1 = _prep(A)
    A2 = _binmm(A_bf, A_bf)
    dinv2, omd2 = _stats(A2)
    A3 = _binmm(A2, A_bf)
    dinv3, omd3 = _stats(A3)

    def conv(h, W, b, a_bf, dinv, omd):
        zs = _linear(h, W, dinv)
        return _prop(a_bf, zs, dinv, omd, b.reshape(1, -1))

    h1 = conv(x, W1_1, b1_1, A_bf, dinv1, omd1)
    h1 = conv(h1, W1_2, b1_2, A_bf, dinv1, omd1)
    h1 = conv(h1, W1_3, b1_3, A_bf, dinv1, omd1)
    h2 = conv(x, W2_1, b2_1, A2, dinv2, omd2)
    h2 = conv(h2, W2_2, b2_2, A2, dinv2, omd2)
    h3 = conv(x, W3_1, b3_1, A3, dinv3, omd3)

    concat = jnp.concatenate([h1, h2, h3], axis=1)
    concat = jnp.pad(concat, ((0, 0), (0, FP - concat.shape[1])))
    partials = _segmax(concat, batch, neginf)

    Wg1p = jnp.pad(Wg1, ((0, FP - Wg1.shape[0]), (0, 0)))
    return _head(partials, Wg1p, bg1.reshape(1, -1), Wg2, bg2.reshape(1, -1))


# speedup vs baseline: 1.2164x; 1.2164x over previous
import functools

import jax
import jax.numpy as jnp
from jax import lax
from jax.experimental import pallas as pl
from jax.experimental.pallas import tpu as pltpu
from jax.experimental.pallas import tpu_sc as plsc

N = 4096
E = 16384
G = 128
FP = 560

_NC = 2
_NS = 16
_NW = _NC * _NS
_ROWS_W = N // _NW
_CH = 16

_mesh = functools.partial(
    plsc.VectorSubcoreMesh, core_axis_name="c", subcore_axis_name="s")


def _build_adj_body(edge_hbm, zeros_hbm, a_hbm, src_v, dst_v, fl_v, chunk_v):
    wid = lax.axis_index("s") * _NC + lax.axis_index("c")
    row0 = wid * _ROWS_W
    pltpu.sync_copy(edge_hbm.at[0], src_v)
    pltpu.sync_copy(edge_hbm.at[1], dst_v)

    def pre(j, off):
        d = dst_v[pl.ds(j * 16, 16)]
        s = src_v[pl.ds(j * 16, 16)]
        m = (d >= row0) & (d < row0 + _ROWS_W)
        fl = (d - row0) * N + s
        plsc.store_compressed(fl_v.at[pl.ds(off, 16)], fl, mask=m)
        return off + plsc.all_reduce_population_count(m)[0]

    cnt = lax.fori_loop(0, E // 16, pre, 0)
    trips = (cnt + 15) // 16
    ones = jnp.full((16,), 1.0, jnp.float32)
    for c in range(_ROWS_W // _CH):
        lo = c * (_CH * N)
        pltpu.sync_copy(zeros_hbm, chunk_v)

        def body(j, carry):
            fl = fl_v[pl.ds(j * 16, 16)]
            lane = lax.iota(jnp.int32, 16)
            idx = fl - lo
            m = ((j * 16 + lane) < cnt) & (idx >= 0) & (idx < _CH * N)
            plsc.store_scatter(chunk_v, [jnp.where(m, idx, 0)], ones, mask=m)
            return carry

        lax.fori_loop(0, trips, body, 0)
        pltpu.sync_copy(chunk_v,
                        a_hbm.at[pl.ds((row0 + c * _CH) * N, _CH * N)])


def _build_adj(edge_index, zeros_chunk):
    return pl.kernel(
        _build_adj_body,
        out_type=jax.ShapeDtypeStruct((N * N,), jnp.float32),
        mesh=_mesh(),
        compiler_params=pltpu.CompilerParams(needs_layout_passes=False),
        scratch_types=[
            pltpu.VMEM((E,), jnp.int32),
            pltpu.VMEM((E,), jnp.int32),
            pltpu.VMEM((E + 16,), jnp.int32),
            pltpu.VMEM((_CH * N,), jnp.float32),
        ],
    )(edge_index, zeros_chunk)


def _segmax_body(concat_hbm, batch_hbm, neginf_hbm, part_hbm,
                 pooled_v, slab_v, batch_v):
    wid = lax.axis_index("s") * _NC + lax.axis_index("c")
    n0 = wid * _ROWS_W
    pltpu.sync_copy(neginf_hbm, pooled_v)
    pltpu.sync_copy(batch_hbm.at[pl.ds(n0, _ROWS_W)],
                    batch_v.at[pl.ds(0, _ROWS_W)])
    for half in range(2):
        pltpu.sync_copy(concat_hbm.at[pl.ds(n0 + half * 64, 64)], slab_v)

        def body(n, carry):
            g = batch_v[pl.ds(half * 64 + n, 16)][0]
            for f in range(FP // 16):
                sl = slab_v[n, pl.ds(f * 16, 16)]
                pv = pooled_v[g, pl.ds(f * 16, 16)]
                pooled_v[g, pl.ds(f * 16, 16)] = jnp.maximum(pv, sl)
            return carry

        lax.fori_loop(0, 64, body, 0)
    pltpu.sync_copy(pooled_v, part_hbm.at[wid])


def _segmax(concat, batch, neginf):
    return pl.kernel(
        _segmax_body,
        out_type=jax.ShapeDtypeStruct((_NW, G, FP), jnp.float32),
        mesh=_mesh(),
        scratch_types=[
            pltpu.VMEM((G, FP), jnp.float32),
            pltpu.VMEM((64, FP), jnp.float32),
            pltpu.VMEM((_ROWS_W + 16,), jnp.int32),
        ],
    )(concat, batch, neginf)


_BM = 256


def _prep_body(a_ref, abf_ref, dinv_ref, omd_ref):
    i = pl.program_id(0)
    a = a_ref[...]
    rowsum = jnp.sum(a, axis=1, keepdims=True)
    r = lax.broadcasted_iota(jnp.int32, a.shape, 0) + i * _BM
    c = lax.broadcasted_iota(jnp.int32, a.shape, 1)
    diag = jnp.sum(jnp.where(r == c, a, 0.0), axis=1, keepdims=True)
    deg = rowsum + 1.0 - diag
    dinv_ref[...] = lax.rsqrt(deg)
    omd_ref[...] = 1.0 - diag
    abf_ref[...] = a.astype(jnp.int8)


def _prep(a):
    return pl.pallas_call(
        _prep_body,
        grid=(N // _BM,),
        in_specs=[pl.BlockSpec((_BM, N), lambda i: (i, 0))],
        out_specs=[
            pl.BlockSpec((_BM, N), lambda i: (i, 0)),
            pl.BlockSpec((_BM, 1), lambda i: (i, 0)),
            pl.BlockSpec((_BM, 1), lambda i: (i, 0)),
        ],
        out_shape=[
            jax.ShapeDtypeStruct((N, N), jnp.int8),
            jax.ShapeDtypeStruct((N, 1), jnp.float32),
            jax.ShapeDtypeStruct((N, 1), jnp.float32),
        ],
    )(a)


def _stats_body(a_ref, dinv_ref, omd_ref):
    i = pl.program_id(0)
    a = a_ref[...].astype(jnp.float32)
    rowsum = jnp.sum(a, axis=1, keepdims=True)
    r = lax.broadcasted_iota(jnp.int32, a.shape, 0) + i * _BM
    c = lax.broadcasted_iota(jnp.int32, a.shape, 1)
    diag = jnp.sum(jnp.where(r == c, a, 0.0), axis=1, keepdims=True)
    deg = rowsum + 1.0 - diag
    dinv_ref[...] = lax.rsqrt(deg)
    omd_ref[...] = 1.0 - diag


def _stats(a_bf):
    return pl.pallas_call(
        _stats_body,
        grid=(N // _BM,),
        in_specs=[pl.BlockSpec((_BM, N), lambda i: (i, 0))],
        out_specs=[
            pl.BlockSpec((_BM, 1), lambda i: (i, 0)),
            pl.BlockSpec((_BM, 1), lambda i: (i, 0)),
        ],
        out_shape=[
            jax.ShapeDtypeStruct((N, 1), jnp.float32),
            jax.ShapeDtypeStruct((N, 1), jnp.float32),
        ],
    )(a_bf)


_MM_BN = 512


def _binmm_body(x_ref, y_ref, o_ref):
    p = jnp.dot(x_ref[...], y_ref[...], preferred_element_type=jnp.int32)
    o_ref[...] = (p > 0).astype(jnp.int8)


def _binmm(x_i8, y_i8):
    return pl.pallas_call(
        _binmm_body,
        grid=(N // _MM_BN,),
        in_specs=[
            pl.BlockSpec((N, N), lambda j: (0, 0)),
            pl.BlockSpec((N, _MM_BN), lambda j: (0, j)),
        ],
        out_specs=pl.BlockSpec((N, _MM_BN), lambda j: (0, j)),
        out_shape=jax.ShapeDtypeStruct((N, N), jnp.int8),
    )(x_i8, y_i8)


def _linear_body(h_ref, w_ref, dinv_ref, hi_ref, lo_ref):
    z = (jnp.dot(h_ref[...], w_ref[...], preferred_element_type=jnp.float32)
         * dinv_ref[...])
    hi = z.astype(jnp.bfloat16)
    hi_ref[...] = hi
    lo_ref[...] = (z - hi.astype(jnp.float32)).astype(jnp.bfloat16)


def _linear(h, w, dinv):
    fin, fout = w.shape
    return pl.pallas_call(
        _linear_body,
        grid=(N // _BM,),
        in_specs=[
            pl.BlockSpec((_BM, fin), lambda i: (i, 0)),
            pl.BlockSpec((fin, fout), lambda i: (0, 0)),
            pl.BlockSpec((_BM, 1), lambda i: (i, 0)),
        ],
        out_specs=[
            pl.BlockSpec((_BM, fout), lambda i: (i, 0)),
            pl.BlockSpec((_BM, fout), lambda i: (i, 0)),
        ],
        out_shape=[
            jax.ShapeDtypeStruct((N, fout), jnp.bfloat16),
            jax.ShapeDtypeStruct((N, fout), jnp.bfloat16),
        ],
    )(h, w, dinv)


def _prop_body(a_ref, zhi_ref, zlo_ref, dinv_ref, omd_ref, b_ref, o_ref):
    i = pl.program_id(0)
    a = a_ref[...].astype(jnp.bfloat16)
    r = lax.broadcasted_iota(jnp.int32, a.shape, 0) + i * _BM
    c = lax.broadcasted_iota(jnp.int32, a.shape, 1)
    d = jnp.where(r == c, omd_ref[...], 0.0)
    a = a + d.astype(jnp.bfloat16)
    acc = (jnp.dot(a, zhi_ref[...], preferred_element_type=jnp.float32)
           + jnp.dot(a, zlo_ref[...], preferred_element_type=jnp.float32))
    o_ref[...] = jnp.maximum(dinv_ref[...] * acc + b_ref[...], 0.0)


def _prop(a_i8, zhi, zlo, dinv, omd, b):
    f = zhi.shape[1]
    return pl.pallas_call(
        _prop_body,
        grid=(N // _BM,),
        in_specs=[
            pl.BlockSpec((_BM, N), lambda i: (i, 0)),
            pl.BlockSpec((N, f), lambda i: (0, 0)),
            pl.BlockSpec((N, f), lambda i: (0, 0)),
            pl.BlockSpec((_BM, 1), lambda i: (i, 0)),
            pl.BlockSpec((_BM, 1), lambda i: (i, 0)),
            pl.BlockSpec((1, f), lambda i: (0, 0)),
        ],
        out_specs=pl.BlockSpec((_BM, f), lambda i: (i, 0)),
        out_shape=jax.ShapeDtypeStruct((N, f), jnp.float32),
    )(a_i8, zhi, zlo, dinv, omd, b)


def _head_body(p_ref, w1_ref, b1_ref, w2_ref, b2_ref, o_ref, acc_ref):
    p = pl.program_id(0)
    blk = p_ref[0]

    @pl.when(p == 0)
    def _():
        acc_ref[...] = blk

    @pl.when(p > 0)
    def _():
        acc_ref[...] = jnp.maximum(acc_ref[...], blk)

    @pl.when(p == pl.num_programs(0) - 1)
    def _():
        t = jnp.dot(acc_ref[...], w1_ref[...],
                    preferred_element_type=jnp.float32) + b1_ref[...]
        t = jnp.maximum(t, 0.0)
        o_ref[...] = jnp.dot(t, w2_ref[...],
                             preferred_element_type=jnp.float32) + b2_ref[...]


def _head(partials, w1p, b1, w2, b2):
    return pl.pallas_call(
        _head_body,
        grid=(_NW,),
        in_specs=[
            pl.BlockSpec((1, G, FP), lambda p: (p, 0, 0)),
            pl.BlockSpec((FP, 512), lambda p: (0, 0)),
            pl.BlockSpec((1, 512), lambda p: (0, 0)),
            pl.BlockSpec((512, G), lambda p: (0, 0)),
            pl.BlockSpec((1, G), lambda p: (0, 0)),
        ],
        out_specs=pl.BlockSpec((G, G), lambda p: (0, 0)),
        out_shape=jax.ShapeDtypeStruct((G, G), jnp.float32),
        scratch_shapes=[pltpu.VMEM((G, FP), jnp.float32)],
    )(partials, w1p, b1, w2, b2)


def kernel(x, edge_index, batch, W1_1, b1_1, W1_2, b1_2, W1_3, b1_3,
           W2_1, b2_1, W2_2, b2_2, W3_1, b3_1, Wg1, bg1, Wg2, bg2):
    zeros_chunk = jnp.zeros((_CH * N,), jnp.float32)
    neginf = jnp.full((G, FP), -jnp.inf, jnp.float32)

    A = _build_adj(edge_index, zeros_chunk).reshape(N, N)
    A_bf, dinv1, omd1 = _prep(A)
    A2 = _binmm(A_bf, A_bf)
    dinv2, omd2 = _stats(A2)
    A3 = _binmm(A2, A_bf)
    dinv3, omd3 = _stats(A3)

    def conv(h, W, b, a_i8, dinv, omd):
        zhi, zlo = _linear(h, W, dinv)
        return _prop(a_i8, zhi, zlo, dinv, omd, b.reshape(1, -1))

    h1 = conv(x, W1_1, b1_1, A_bf, dinv1, omd1)
    h1 = conv(h1, W1_2, b1_2, A_bf, dinv1, omd1)
    h1 = conv(h1, W1_3, b1_3, A_bf, dinv1, omd1)
    h2 = conv(x, W2_1, b2_1, A2, dinv2, omd2)
    h2 = conv(h2, W2_2, b2_2, A2, dinv2, omd2)
    h3 = conv(x, W3_1, b3_1, A3, dinv3, omd3)

    concat = jnp.concatenate([h1, h2, h3], axis=1)
    concat = jnp.pad(concat, ((0, 0), (0, FP - concat.shape[1])))
    partials = _segmax(concat, batch, neginf)

    Wg1p = jnp.pad(Wg1, ((0, FP - Wg1.shape[0]), (0, 0)))
    return _head(partials, Wg1p, bg1.reshape(1, -1), Wg2, bg2.reshape(1, -1))


# speedup vs baseline: 1.3200x; 1.0852x over previous
import functools

import jax
import jax.numpy as jnp
from jax import lax
from jax.experimental import pallas as pl
from jax.experimental.pallas import tpu as pltpu
from jax.experimental.pallas import tpu_sc as plsc

N = 4096
E = 16384
G = 128
FP = 560

_NC = 2
_NS = 16
_NW = _NC * _NS
_ROWS_W = N // _NW
_CH = 16

_mesh = functools.partial(
    plsc.VectorSubcoreMesh, core_axis_name="c", subcore_axis_name="s")


def _build_adj_body(edge_hbm, zeros_hbm, a_hbm, src_v, dst_v, fl_v, chunk_v):
    wid = lax.axis_index("s") * _NC + lax.axis_index("c")
    row0 = wid * _ROWS_W
    pltpu.sync_copy(edge_hbm.at[0], src_v)
    pltpu.sync_copy(edge_hbm.at[1], dst_v)

    def pre(j, off):
        d = dst_v[pl.ds(j * 16, 16)]
        s = src_v[pl.ds(j * 16, 16)]
        m = (d >= row0) & (d < row0 + _ROWS_W)
        fl = (d - row0) * N + s
        plsc.store_compressed(fl_v.at[pl.ds(off, 16)], fl, mask=m)
        return off + plsc.all_reduce_population_count(m)[0]

    cnt = lax.fori_loop(0, E // 16, pre, 0)
    trips = (cnt + 15) // 16
    ones = jnp.full((16,), 1.0, jnp.float32)
    zeros = jnp.zeros((16,), jnp.float32)
    pltpu.sync_copy(zeros_hbm, chunk_v)
    for c in range(_ROWS_W // _CH):
        lo = c * (_CH * N)

        def body(j, carry, lo=lo, vals=ones):
            fl = fl_v[pl.ds(j * 16, 16)]
            lane = lax.iota(jnp.int32, 16)
            idx = fl - lo
            m = ((j * 16 + lane) < cnt) & (idx >= 0) & (idx < _CH * N)
            plsc.store_scatter(chunk_v, [jnp.where(m, idx, 0)], vals, mask=m)
            return carry

        lax.fori_loop(0, trips, body, 0)
        pltpu.sync_copy(chunk_v,
                        a_hbm.at[pl.ds((row0 + c * _CH) * N, _CH * N)])
        if c + 1 < _ROWS_W // _CH:
            lax.fori_loop(
                0, trips,
                functools.partial(body, lo=lo, vals=zeros), 0)


def _build_adj(edge_index, zeros_chunk):
    return pl.kernel(
        _build_adj_body,
        out_type=jax.ShapeDtypeStruct((N * N,), jnp.float32),
        mesh=_mesh(),
        compiler_params=pltpu.CompilerParams(needs_layout_passes=False),
        scratch_types=[
            pltpu.VMEM((E,), jnp.int32),
            pltpu.VMEM((E,), jnp.int32),
            pltpu.VMEM((E + 16,), jnp.int32),
            pltpu.VMEM((_CH * N,), jnp.float32),
        ],
    )(edge_index, zeros_chunk)


def _segmax_body(concat_hbm, batch_hbm, neginf_hbm, part_hbm,
                 pooled_v, slab_v, batch_v):
    wid = lax.axis_index("s") * _NC + lax.axis_index("c")
    n0 = wid * _ROWS_W
    pltpu.sync_copy(neginf_hbm, pooled_v)
    pltpu.sync_copy(batch_hbm.at[pl.ds(n0, _ROWS_W)],
                    batch_v.at[pl.ds(0, _ROWS_W)])
    for half in range(2):
        pltpu.sync_copy(concat_hbm.at[pl.ds(n0 + half * 64, 64)], slab_v)

        def body(n, carry):
            g = batch_v[pl.ds(half * 64 + n, 16)][0]
            for f in range(FP // 16):
                sl = slab_v[n, pl.ds(f * 16, 16)]
                pv = pooled_v[g, pl.ds(f * 16, 16)]
                pooled_v[g, pl.ds(f * 16, 16)] = jnp.maximum(pv, sl)
            return carry

        lax.fori_loop(0, 64, body, 0)
    pltpu.sync_copy(pooled_v, part_hbm.at[wid])


def _segmax(concat, batch, neginf):
    return pl.kernel(
        _segmax_body,
        out_type=jax.ShapeDtypeStruct((_NW, G, FP), jnp.float32),
        mesh=_mesh(),
        scratch_types=[
            pltpu.VMEM((G, FP), jnp.float32),
            pltpu.VMEM((64, FP), jnp.float32),
            pltpu.VMEM((_ROWS_W + 16,), jnp.int32),
        ],
    )(concat, batch, neginf)


_BM = 256


def _prep_body(a_ref, abf_ref, dinv_ref, omd_ref):
    i = pl.program_id(0)
    a = a_ref[...]
    rowsum = jnp.sum(a, axis=1, keepdims=True)
    r = lax.broadcasted_iota(jnp.int32, a.shape, 0) + i * _BM
    c = lax.broadcasted_iota(jnp.int32, a.shape, 1)
    diag = jnp.sum(jnp.where(r == c, a, 0.0), axis=1, keepdims=True)
    deg = rowsum + 1.0 - diag
    dinv_ref[...] = lax.rsqrt(deg)
    omd_ref[...] = 1.0 - diag
    abf_ref[...] = a.astype(jnp.int8)


def _prep(a):
    return pl.pallas_call(
        _prep_body,
        grid=(N // _BM,),
        in_specs=[pl.BlockSpec((_BM, N), lambda i: (i, 0))],
        out_specs=[
            pl.BlockSpec((_BM, N), lambda i: (i, 0)),
            pl.BlockSpec((_BM, 1), lambda i: (i, 0)),
            pl.BlockSpec((_BM, 1), lambda i: (i, 0)),
        ],
        out_shape=[
            jax.ShapeDtypeStruct((N, N), jnp.int8),
            jax.ShapeDtypeStruct((N, 1), jnp.float32),
            jax.ShapeDtypeStruct((N, 1), jnp.float32),
        ],
    )(a)


_MM_BN = 512


def _binmm_body(x_ref, y_ref, o_ref, dinv_ref, omd_ref, rs_ref, dg_ref):
    j = pl.program_id(0)
    p = jnp.dot(x_ref[...], y_ref[...], preferred_element_type=jnp.int32)
    pb = p > 0
    o_ref[...] = pb.astype(jnp.int8)
    pf = pb.astype(jnp.float32)
    prs = jnp.sum(pf, axis=1, keepdims=True)
    r = lax.broadcasted_iota(jnp.int32, pf.shape, 0)
    c = lax.broadcasted_iota(jnp.int32, pf.shape, 1) + j * _MM_BN
    pdg = jnp.sum(jnp.where(r == c, pf, 0.0), axis=1, keepdims=True)

    @pl.when(j == 0)
    def _():
        rs_ref[...] = prs
        dg_ref[...] = pdg

    @pl.when(j > 0)
    def _():
        rs_ref[...] += prs
        dg_ref[...] += pdg

    @pl.when(j == pl.num_programs(0) - 1)
    def _():
        deg = rs_ref[...] + 1.0 - dg_ref[...]
        dinv_ref[...] = lax.rsqrt(deg)
        omd_ref[...] = 1.0 - dg_ref[...]


def _binmm(x_i8, y_i8):
    return pl.pallas_call(
        _binmm_body,
        grid=(N // _MM_BN,),
        in_specs=[
            pl.BlockSpec((N, N), lambda j: (0, 0)),
            pl.BlockSpec((N, _MM_BN), lambda j: (0, j)),
        ],
        out_specs=[
            pl.BlockSpec((N, _MM_BN), lambda j: (0, j)),
            pl.BlockSpec((N, 1), lambda j: (0, 0)),
            pl.BlockSpec((N, 1), lambda j: (0, 0)),
        ],
        out_shape=[
            jax.ShapeDtypeStruct((N, N), jnp.int8),
            jax.ShapeDtypeStruct((N, 1), jnp.float32),
            jax.ShapeDtypeStruct((N, 1), jnp.float32),
        ],
        scratch_shapes=[
            pltpu.VMEM((N, 1), jnp.float32),
            pltpu.VMEM((N, 1), jnp.float32),
        ],
    )(x_i8, y_i8)


def _conv_body(h_ref, w_ref, a_ref, dinv_ref, omd_ref, b_ref, o_ref,
               zhi_ref, zlo_ref):
    ph = pl.program_id(0)
    i = pl.program_id(1)

    @pl.when(ph == 0)
    def _():
        z = (jnp.dot(h_ref[...], w_ref[...],
                     preferred_element_type=jnp.float32) * dinv_ref[...])
        hi = z.astype(jnp.bfloat16)
        zhi_ref[pl.ds(i * _BM, _BM), :] = hi
        zlo_ref[pl.ds(i * _BM, _BM), :] = (
            z - hi.astype(jnp.float32)).astype(jnp.bfloat16)

    @pl.when(ph == 1)
    def _():
        a = a_ref[...].astype(jnp.bfloat16)
        r = lax.broadcasted_iota(jnp.int32, a.shape, 0) + i * _BM
        c = lax.broadcasted_iota(jnp.int32, a.shape, 1)
        d = jnp.where(r == c, omd_ref[...], 0.0)
        a = a + d.astype(jnp.bfloat16)
        acc = (jnp.dot(a, zhi_ref[...], preferred_element_type=jnp.float32)
               + jnp.dot(a, zlo_ref[...], preferred_element_type=jnp.float32))
        o_ref[...] = jnp.maximum(dinv_ref[...] * acc + b_ref[...], 0.0)


def _conv(h, w, a_i8, dinv, omd, b):
    fin, fout = w.shape
    return pl.pallas_call(
        _conv_body,
        grid=(2, N // _BM),
        in_specs=[
            pl.BlockSpec((_BM, fin),
                         lambda p, i: (jnp.where(p == 0, i, 0), 0)),
            pl.BlockSpec((fin, fout), lambda p, i: (0, 0)),
            pl.BlockSpec((_BM, N),
                         lambda p, i: (jnp.where(p == 1, i, 0), 0)),
            pl.BlockSpec((_BM, 1), lambda p, i: (i, 0)),
            pl.BlockSpec((_BM, 1), lambda p, i: (i, 0)),
            pl.BlockSpec((1, fout), lambda p, i: (0, 0)),
        ],
        out_specs=pl.BlockSpec((_BM, fout),
                               lambda p, i: (jnp.where(p == 1, i, 0), 0)),
        out_shape=jax.ShapeDtypeStruct((N, fout), jnp.float32),
        scratch_shapes=[
            pltpu.VMEM((N, fout), jnp.bfloat16),
            pltpu.VMEM((N, fout), jnp.bfloat16),
        ],
    )(h, w, a_i8, dinv, omd, b)


def _head_body(p_ref, w1_ref, b1_ref, w2_ref, b2_ref, o_ref, acc_ref):
    p = pl.program_id(0)
    blk = p_ref[0]

    @pl.when(p == 0)
    def _():
        acc_ref[...] = blk

    @pl.when(p > 0)
    def _():
        acc_ref[...] = jnp.maximum(acc_ref[...], blk)

    @pl.when(p == pl.num_programs(0) - 1)
    def _():
        t = jnp.dot(acc_ref[...], w1_ref[...],
                    preferred_element_type=jnp.float32) + b1_ref[...]
        t = jnp.maximum(t, 0.0)
        o_ref[...] = jnp.dot(t, w2_ref[...],
                             preferred_element_type=jnp.float32) + b2_ref[...]


def _head(partials, w1p, b1, w2, b2):
    return pl.pallas_call(
        _head_body,
        grid=(_NW,),
        in_specs=[
            pl.BlockSpec((1, G, FP), lambda p: (p, 0, 0)),
            pl.BlockSpec((FP, 512), lambda p: (0, 0)),
            pl.BlockSpec((1, 512), lambda p: (0, 0)),
            pl.BlockSpec((512, G), lambda p: (0, 0)),
            pl.BlockSpec((1, G), lambda p: (0, 0)),
        ],
        out_specs=pl.BlockSpec((G, G), lambda p: (0, 0)),
        out_shape=jax.ShapeDtypeStruct((G, G), jnp.float32),
        scratch_shapes=[pltpu.VMEM((G, FP), jnp.float32)],
    )(partials, w1p, b1, w2, b2)


def kernel(x, edge_index, batch, W1_1, b1_1, W1_2, b1_2, W1_3, b1_3,
           W2_1, b2_1, W2_2, b2_2, W3_1, b3_1, Wg1, bg1, Wg2, bg2):
    zeros_chunk = jnp.zeros((_CH * N,), jnp.float32)
    neginf = jnp.full((G, FP), -jnp.inf, jnp.float32)

    A = _build_adj(edge_index, zeros_chunk).reshape(N, N)
    A_i8, dinv1, omd1 = _prep(A)
    A2, dinv2, omd2 = _binmm(A_i8, A_i8)
    A3, dinv3, omd3 = _binmm(A2, A_i8)

    def conv(h, W, b, a_i8, dinv, omd):
        return _conv(h, W, a_i8, dinv, omd, b.reshape(1, -1))

    h1 = conv(x, W1_1, b1_1, A_i8, dinv1, omd1)
    h1 = conv(h1, W1_2, b1_2, A_i8, dinv1, omd1)
    h1 = conv(h1, W1_3, b1_3, A_i8, dinv1, omd1)
    h2 = conv(x, W2_1, b2_1, A2, dinv2, omd2)
    h2 = conv(h2, W2_2, b2_2, A2, dinv2, omd2)
    h3 = conv(x, W3_1, b3_1, A3, dinv3, omd3)

    concat = jnp.concatenate([h1, h2, h3], axis=1)
    concat = jnp.pad(concat, ((0, 0), (0, FP - concat.shape[1])))
    partials = _segmax(concat, batch, neginf)

    Wg1p = jnp.pad(Wg1, ((0, FP - Wg1.shape[0]), (0, 0)))
    return _head(partials, Wg1p, bg1.reshape(1, -1), Wg2, bg2.reshape(1, -1))
